# Initial kernel scaffold; baseline (speedup 1.0000x reference)
import jax
import jax.numpy as jnp
from jax.experimental import pallas as pl


def kernel(x, emb, W1, b1, W2, b2):
    raise NotImplementedError("write your pallas kernel here")


# speedup vs baseline: 2.0526x; 2.0526x over previous
import functools

import jax
import jax.numpy as jnp
from jax import lax
from jax.experimental import pallas as pl
from jax.experimental.pallas import tpu as pltpu
from jax.experimental.pallas import tpu_sc as plsc

_B = 4096
_H = 200
_D = 32
_NW = 32
_BPW = _B // _NW
_C0 = 128
_C1 = _H - _C0


def _pool_body(x_hbm, emb_hbm, out_hbm, idx_v, rows_v, acc_v, sem):
    c = lax.axis_index("c")
    s = lax.axis_index("s")
    wid = s * 2 + c
    base = wid * _BPW
    pltpu.sync_copy(x_hbm.at[pl.ds(base, _BPW)], idx_v)

    def one_row(b, carry):
        h1 = pltpu.async_copy(emb_hbm.at[idx_v.at[b, pl.ds(0, _C0)]],
                              rows_v.at[pl.ds(0, _C0)], sem)
        h2 = pltpu.async_copy(emb_hbm.at[idx_v.at[b, pl.ds(_C0, _C1)]],
                              rows_v.at[pl.ds(_C0, _C1)], sem)
        h1.wait()
        h2.wait()

        def red(j, acc):
            a0, a1 = acc
            return (a0 + rows_v[j, 0:16], a1 + rows_v[j, 16:32])

        z = jnp.zeros((16,), jnp.float32)
        a0, a1 = lax.fori_loop(0, _H, red, (z, z), unroll=8)
        acc_v[b, 0:16] = a0
        acc_v[b, 16:32] = a1
        return carry

    lax.fori_loop(0, _BPW, one_row, 0)
    pltpu.sync_copy(acc_v, out_hbm.at[pl.ds(base, _BPW)])


_pool = functools.partial(
    pl.kernel,
    out_type=jax.ShapeDtypeStruct((_B, _D), jnp.float32),
    mesh=plsc.VectorSubcoreMesh(core_axis_name="c", subcore_axis_name="s"),
    scratch_types=[
        pltpu.VMEM((_BPW, _H), jnp.int32),
        pltpu.VMEM((_H, _D), jnp.float32),
        pltpu.VMEM((_BPW, _D), jnp.float32),
        pltpu.SemaphoreType.DMA,
    ],
    compiler_params=pltpu.CompilerParams(use_tc_tiling_on_sc=False),
)(_pool_body)


def _mlp_body(p_ref, w1_ref, b1_ref, w2_ref, b2_ref, o_ref):
    h = p_ref[...] * (1.0 / _H)
    h = jnp.dot(h, w1_ref[...], preferred_element_type=jnp.float32) + b1_ref[...]
    h = jnp.maximum(h, 0.0)
    o_ref[...] = jnp.dot(h, w2_ref[...], preferred_element_type=jnp.float32) + b2_ref[...]


def kernel(x, emb, W1, b1, W2, b2):
    pooled = _pool(x, emb)
    w2p = jnp.zeros((_D, 128), jnp.float32).at[:, :10].set(W2)
    b2p = jnp.zeros((1, 128), jnp.float32).at[:, :10].set(b2)
    out = pl.pallas_call(
        _mlp_body,
        out_shape=jax.ShapeDtypeStruct((_B, 128), jnp.float32),
    )(pooled, W1, b1.reshape(1, _D), w2p, b2p)
    return out[:, :10]


# speedup vs baseline: 2.4206x; 1.1793x over previous
import functools

import jax
import jax.numpy as jnp
from jax import lax
from jax.experimental import pallas as pl
from jax.experimental.pallas import tpu as pltpu
from jax.experimental.pallas import tpu_sc as plsc

_B = 4096
_H = 200
_D = 32
_NW = 32
_BPW = _B // _NW
_C0 = 128
_C1 = _H - _C0


_NBUF = 4


def _pool_body(x_hbm, emb_hbm, out_hbm, idx_v, rows_v, acc_v, *sems):
    c = lax.axis_index("c")
    s = lax.axis_index("s")
    wid = s * 2 + c
    base = wid * _BPW
    pltpu.sync_copy(x_hbm.at[pl.ds(base, _BPW)], idx_v)

    def issue(b, buf):
        pltpu.async_copy(emb_hbm.at[idx_v.at[b, pl.ds(0, _C0)]],
                         rows_v.at[buf, pl.ds(0, _C0)], sems[buf])
        pltpu.async_copy(emb_hbm.at[idx_v.at[b, pl.ds(_C0, _C1)]],
                         rows_v.at[buf, pl.ds(_C0, _C1)], sems[buf])

    def drain(b, buf):
        pltpu.make_async_copy(emb_hbm.at[idx_v.at[b, pl.ds(0, _C0)]],
                              rows_v.at[buf, pl.ds(0, _C0)], sems[buf]).wait()
        pltpu.make_async_copy(emb_hbm.at[idx_v.at[b, pl.ds(_C0, _C1)]],
                              rows_v.at[buf, pl.ds(_C0, _C1)], sems[buf]).wait()

    for p in range(_NBUF):
        issue(p, p)

    def one_group(t, carry):
        for p in range(_NBUF):
            b = _NBUF * t + p
            drain(b, p)

            def red(j, acc):
                a0, a1 = acc
                return (a0 + rows_v[p, j, 0:16], a1 + rows_v[p, j, 16:32])

            z = jnp.zeros((16,), jnp.float32)
            a0, a1 = lax.fori_loop(0, _H, red, (z, z), unroll=8)
            acc_v[b, 0:16] = a0
            acc_v[b, 16:32] = a1

            @pl.when(b + _NBUF < _BPW)
            def _():
                issue(b + _NBUF, p)
        return carry

    lax.fori_loop(0, _BPW // _NBUF, one_group, 0)
    pltpu.sync_copy(acc_v, out_hbm.at[pl.ds(base, _BPW)])


_pool = functools.partial(
    pl.kernel,
    out_type=jax.ShapeDtypeStruct((_B, _D), jnp.float32),
    mesh=plsc.VectorSubcoreMesh(core_axis_name="c", subcore_axis_name="s"),
    scratch_types=[
        pltpu.VMEM((_BPW, _H), jnp.int32),
        pltpu.VMEM((_NBUF, _H, _D), jnp.float32),
        pltpu.VMEM((_BPW, _D), jnp.float32),
    ] + [pltpu.SemaphoreType.DMA] * _NBUF,
    compiler_params=pltpu.CompilerParams(use_tc_tiling_on_sc=False),
)(_pool_body)


def _mlp_body(p_ref, w1_ref, b1_ref, w2_ref, b2_ref, o_ref):
    h = p_ref[...] * (1.0 / _H)
    h = jnp.dot(h, w1_ref[...], preferred_element_type=jnp.float32) + b1_ref[...]
    h = jnp.maximum(h, 0.0)
    o_ref[...] = jnp.dot(h, w2_ref[...], preferred_element_type=jnp.float32) + b2_ref[...]


def kernel(x, emb, W1, b1, W2, b2):
    pooled = _pool(x, emb)
    w2p = jnp.zeros((_D, 128), jnp.float32).at[:, :10].set(W2)
    b2p = jnp.zeros((1, 128), jnp.float32).at[:, :10].set(b2)
    out = pl.pallas_call(
        _mlp_body,
        out_shape=jax.ShapeDtypeStruct((_B, 128), jnp.float32),
    )(pooled, W1, b1.reshape(1, _D), w2p, b2p)
    return out[:, :10]


# speedup vs baseline: 2.4225x; 1.0008x over previous
import functools

import jax
import jax.numpy as jnp
from jax import lax
from jax.experimental import pallas as pl
from jax.experimental.pallas import tpu as pltpu
from jax.experimental.pallas import tpu_sc as plsc

_B = 4096
_H = 200
_D = 32
_NW = 32
_BPW = _B // _NW
_C0 = 128
_C1 = _H - _C0


_NBUF = 4


def _pool_body(x_hbm, emb_hbm, out_hbm, idx_v, rows_v, acc_v, *sems):
    c = lax.axis_index("c")
    s = lax.axis_index("s")
    wid = s * 2 + c
    base = wid * _BPW
    pltpu.sync_copy(x_hbm.at[pl.ds(base, _BPW)], idx_v)

    def issue(b, buf):
        pltpu.async_copy(emb_hbm.at[idx_v.at[b, pl.ds(0, _C0)]],
                         rows_v.at[buf, pl.ds(0, _C0)], sems[buf])
        pltpu.async_copy(emb_hbm.at[idx_v.at[b, pl.ds(_C0, _C1)]],
                         rows_v.at[buf, pl.ds(_C0, _C1)], sems[buf])

    def drain(b, buf):
        pltpu.make_async_copy(emb_hbm.at[idx_v.at[b, pl.ds(0, _C0)]],
                              rows_v.at[buf, pl.ds(0, _C0)], sems[buf]).wait()
        pltpu.make_async_copy(emb_hbm.at[idx_v.at[b, pl.ds(_C0, _C1)]],
                              rows_v.at[buf, pl.ds(_C0, _C1)], sems[buf]).wait()

    for p in range(_NBUF):
        issue(p, p)

    def one_group(t, carry):
        for p in range(_NBUF):
            b = _NBUF * t + p
            drain(b, p)

            def red(i, acc):
                new = []
                for k in range(4):
                    a0, a1 = acc[2 * k], acc[2 * k + 1]
                    j = 4 * i + k
                    new.append(a0 + rows_v[p, j, 0:16])
                    new.append(a1 + rows_v[p, j, 16:32])
                return tuple(new)

            z = jnp.zeros((16,), jnp.float32)
            acc = lax.fori_loop(0, _H // 4, red, (z,) * 8, unroll=4)
            acc_v[b, 0:16] = acc[0] + acc[2] + (acc[4] + acc[6])
            acc_v[b, 16:32] = acc[1] + acc[3] + (acc[5] + acc[7])

            @pl.when(b + _NBUF < _BPW)
            def _():
                issue(b + _NBUF, p)
        return carry

    lax.fori_loop(0, _BPW // _NBUF, one_group, 0)
    pltpu.sync_copy(acc_v, out_hbm.at[pl.ds(base, _BPW)])


_pool = functools.partial(
    pl.kernel,
    out_type=jax.ShapeDtypeStruct((_B, _D), jnp.float32),
    mesh=plsc.VectorSubcoreMesh(core_axis_name="c", subcore_axis_name="s"),
    scratch_types=[
        pltpu.VMEM((_BPW, _H), jnp.int32),
        pltpu.VMEM((_NBUF, _H, _D), jnp.float32),
        pltpu.VMEM((_BPW, _D), jnp.float32),
    ] + [pltpu.SemaphoreType.DMA] * _NBUF,
    compiler_params=pltpu.CompilerParams(use_tc_tiling_on_sc=False),
)(_pool_body)


def _mlp_body(p_ref, w1_ref, b1_ref, w2_ref, b2_ref, o_ref):
    h = p_ref[...] * (1.0 / _H)
    h = jnp.dot(h, w1_ref[...], preferred_element_type=jnp.float32) + b1_ref[...]
    h = jnp.maximum(h, 0.0)
    o_ref[...] = jnp.dot(h, w2_ref[...], preferred_element_type=jnp.float32) + b2_ref[...]


def kernel(x, emb, W1, b1, W2, b2):
    pooled = _pool(x, emb)
    w2p = jnp.zeros((_D, 128), jnp.float32).at[:, :10].set(W2)
    b2p = jnp.zeros((1, 128), jnp.float32).at[:, :10].set(b2)
    out = pl.pallas_call(
        _mlp_body,
        out_shape=jax.ShapeDtypeStruct((_B, 128), jnp.float32),
    )(pooled, W1, b1.reshape(1, _D), w2p, b2p)
    return out[:, :10]
